# Initial kernel scaffold; baseline (speedup 1.0000x reference)
import jax
import jax.numpy as jnp
from jax.experimental import pallas as pl


def kernel(LM_embedding, x_numerical, x_categorical, edge_index, edge_type, params):
    raise NotImplementedError("write your pallas kernel here")


# speedup vs baseline: 1.0000x; 1.0000x over previous
import jax
import jax.numpy as jnp
from jax.experimental import pallas as pl

HEADS = 4
SEM_HEADS = 4
N_LAYERS = 2
N_REL = 2


def _conv(x, ei, emask, Wq, bq, Wk, bk, Wv, bv, Ws, bs):
    n, h = x.shape
    dh = h // HEADS
    src, dst = ei[0], ei[1]
    q = (x @ Wq + bq).reshape(n, HEADS, dh)
    k = (x @ Wk + bk).reshape(n, HEADS, dh)
    v = (x @ Wv + bv).reshape(n, HEADS, dh)
    logits = (q[dst] * k[src]).sum(-1) / jnp.sqrt(float(dh))
    logits = jnp.where(emask[:, None], logits, -jnp.inf)
    m = jax.ops.segment_max(logits, dst, num_segments=n)
    m = jnp.where(jnp.isfinite(m), m, 0.0)
    ex = jnp.exp(logits - m[dst])
    denom = jax.ops.segment_sum(ex, dst, num_segments=n) + 1e-16
    alpha = ex / denom[dst]
    agg = jax.ops.segment_sum(v[src] * alpha[:, :, None], dst, num_segments=n)
    return agg.reshape(n, h) + x @ Ws + bs


def kernel(LM_embedding, x_numerical, x_categorical, edge_index, edge_type, params):
    act = jax.nn.relu
    h_lm = act(LM_embedding @ params["W_lm"] + params["b_lm"])
    h_num = act(x_numerical @ params["W_num"] + params["b_num"])
    h_cat = act(x_categorical @ params["W_cat"] + params["b_cat"])
    x = jnp.concatenate([h_lm, h_num, h_cat], axis=-1)
    x = act(x @ params["W_mlp"] + params["b_mlp"])
    for i in range(N_LAYERS):
        zs = []
        for r in range(N_REL):
            zs.append(_conv(
                x, edge_index, edge_type == r,
                params["Wq"][i, r], params["bq"][i, r],
                params["Wk"][i, r], params["bk"][i, r],
                params["Wv"][i, r], params["bv"][i, r],
                params["Wskip"][i, r], params["bskip"][i, r]))
        z = jnp.stack(zs, axis=0)
        proj = jnp.tanh(z @ params["Wsem"][i] + params["bsem"][i])
        score = jnp.einsum("rnh,hs->rs", proj, params["asem"][i]) / float(z.shape[1])
        beta = jax.nn.softmax(score, axis=0)
        x = jnp.einsum("rs,rnh->nh", beta, z) / float(SEM_HEADS)
        x = act(x)
    x = act(x @ params["W_pool"] + params["b_pool"])
    return x @ params["W_out"] + params["b_out"]


# speedup vs baseline: 3.8792x; 3.8791x over previous
import functools

import jax
import jax.numpy as jnp
from jax import lax
from jax.experimental import pallas as pl
from jax.experimental.pallas import tpu as pltpu
from jax.experimental.pallas import tpu_sc as plsc

N_LAYERS = 2
N_REL = 2
HEADS = 4
SEM_HEADS = 4
DH = 32
BM = 1000


def _enc_body(lm, xn, xc, wlm, blm, wnum, bnum, wcat, bcat, wmlp, bmlp, o):
    r = lambda t: jnp.maximum(t, 0.0)
    hl = r(jnp.dot(lm[...], wlm[...], preferred_element_type=jnp.float32) + blm[...])
    hn = r(jnp.dot(xn[...], wnum[...], preferred_element_type=jnp.float32) + bnum[...])
    hc = r(jnp.dot(xc[...], wcat[...], preferred_element_type=jnp.float32) + bcat[...])
    acc = jnp.dot(hl, wmlp[0:64, :], preferred_element_type=jnp.float32)
    acc += jnp.dot(hn, wmlp[64:96, :], preferred_element_type=jnp.float32)
    acc += jnp.dot(hc, wmlp[96:128, :], preferred_element_type=jnp.float32)
    o[...] = r(acc + bmlp[...])


def _encoder(LM, xn, xc, p):
    n = LM.shape[0]
    grid = n // BM
    return pl.pallas_call(
        _enc_body,
        grid=(grid,),
        in_specs=[
            pl.BlockSpec((BM, LM.shape[1]), lambda i: (i, 0)),
            pl.BlockSpec((BM, xn.shape[1]), lambda i: (i, 0)),
            pl.BlockSpec((BM, xc.shape[1]), lambda i: (i, 0)),
            pl.BlockSpec(p["W_lm"].shape, lambda i: (0, 0)),
            pl.BlockSpec(p["b_lm"].shape, lambda i: (0,)),
            pl.BlockSpec(p["W_num"].shape, lambda i: (0, 0)),
            pl.BlockSpec(p["b_num"].shape, lambda i: (0,)),
            pl.BlockSpec(p["W_cat"].shape, lambda i: (0, 0)),
            pl.BlockSpec(p["b_cat"].shape, lambda i: (0,)),
            pl.BlockSpec(p["W_mlp"].shape, lambda i: (0, 0)),
            pl.BlockSpec(p["b_mlp"].shape, lambda i: (0,)),
        ],
        out_specs=pl.BlockSpec((BM, 128), lambda i: (i, 0)),
        out_shape=jax.ShapeDtypeStruct((n, 128), jnp.float32),
    )(LM, xn, xc, p["W_lm"], p["b_lm"], p["W_num"], p["b_num"],
      p["W_cat"], p["b_cat"], p["W_mlp"], p["b_mlp"])


def _proj_body(x, wq, bq, wk, bk, wv, bv, ws, bs, qo, ko, vo, so):
    xv = x[...]
    for r in range(N_REL):
        qo[r] = jnp.dot(xv, wq[r], preferred_element_type=jnp.float32) + bq[r]
        ko[r] = jnp.dot(xv, wk[r], preferred_element_type=jnp.float32) + bk[r]
        vo[r] = jnp.dot(xv, wv[r], preferred_element_type=jnp.float32) + bv[r]
        so[r] = jnp.dot(xv, ws[r], preferred_element_type=jnp.float32) + bs[r]


def _projections(x, wq, bq, wk, bk, wv, bv, ws, bs):
    n = x.shape[0]
    grid = n // BM
    wspec = pl.BlockSpec((N_REL, 128, 128), lambda i: (0, 0, 0))
    bspec = pl.BlockSpec((N_REL, 128), lambda i: (0, 0))
    ospec = pl.BlockSpec((N_REL, BM, 128), lambda i: (0, i, 0))
    oshape = jax.ShapeDtypeStruct((N_REL, n, 128), jnp.float32)
    return pl.pallas_call(
        _proj_body,
        grid=(grid,),
        in_specs=[pl.BlockSpec((BM, 128), lambda i: (i, 0)),
                  wspec, bspec, wspec, bspec, wspec, bspec, wspec, bspec],
        out_specs=[ospec, ospec, ospec, ospec],
        out_shape=[oshape, oshape, oshape, oshape],
    )(x, wq, bq, wk, bk, wv, bv, ws, bs)


def _combine_body(num, den, bmat, skip, wsem, bsem, z, ps):
    i = pl.program_id(0)

    @pl.when(i == 0)
    def _init():
        ps[...] = jnp.zeros_like(ps)

    sums = []
    for r in range(N_REL):
        db = jnp.dot(den[r], bmat[...], preferred_element_type=jnp.float32)
        zr = num[r] / (db + 1e-16) + skip[r]
        z[r] = zr
        t = jnp.tanh(jnp.dot(zr, wsem[...], preferred_element_type=jnp.float32)
                     + bsem[...])
        sums.append(jnp.sum(t, axis=0))
    ps[...] += jnp.stack(sums, axis=0)


def _combine(num, den, bmat, skip, wsem, bsem):
    n = num.shape[1]
    grid = n // BM
    return pl.pallas_call(
        _combine_body,
        grid=(grid,),
        in_specs=[pl.BlockSpec((N_REL, BM, 128), lambda i: (0, i, 0)),
                  pl.BlockSpec((N_REL, BM, 16), lambda i: (0, i, 0)),
                  pl.BlockSpec((16, 128), lambda i: (0, 0)),
                  pl.BlockSpec((N_REL, BM, 128), lambda i: (0, i, 0)),
                  pl.BlockSpec((128, 128), lambda i: (0, 0)),
                  pl.BlockSpec((128,), lambda i: (0,))],
        out_specs=[pl.BlockSpec((N_REL, BM, 128), lambda i: (0, i, 0)),
                   pl.BlockSpec((N_REL, 128), lambda i: (0, 0))],
        out_shape=[jax.ShapeDtypeStruct((N_REL, n, 128), jnp.float32),
                   jax.ShapeDtypeStruct((N_REL, 128), jnp.float32)],
    )(num, den, bmat, skip, wsem, bsem)


def _mix_body(z, ps, asem, o):
    n_total = o.shape[0] * pl.num_programs(0)
    score = jnp.dot(ps[...], asem[...],
                    preferred_element_type=jnp.float32) / float(n_total)
    m = jnp.max(score, axis=0, keepdims=True)
    e = jnp.exp(score - m)
    beta = e / jnp.sum(e, axis=0, keepdims=True)
    c0 = jnp.sum(beta[0:1, :]) / float(SEM_HEADS)
    c1 = jnp.sum(beta[1:2, :]) / float(SEM_HEADS)
    o[...] = jnp.maximum(z[0] * c0 + z[1] * c1, 0.0)


def _mix(z, ps, asem):
    n = z.shape[1]
    grid = n // BM
    return pl.pallas_call(
        _mix_body,
        grid=(grid,),
        in_specs=[pl.BlockSpec((N_REL, BM, 128), lambda i: (0, i, 0)),
                  pl.BlockSpec((N_REL, 128), lambda i: (0, 0)),
                  pl.BlockSpec((128, SEM_HEADS), lambda i: (0, 0))],
        out_specs=pl.BlockSpec((BM, 128), lambda i: (i, 0)),
        out_shape=jax.ShapeDtypeStruct((n, 128), jnp.float32),
    )(z, ps, asem)


def _head_body(x, wp, bp, wo, bo, o):
    h = jnp.maximum(jnp.dot(x[...], wp[...], preferred_element_type=jnp.float32)
                    + bp[...], 0.0)
    o[...] = jnp.dot(h, wo[...], preferred_element_type=jnp.float32) + bo[...]


def _head(x, wp, bp, wo, bo):
    n = x.shape[0]
    grid = n // BM
    return pl.pallas_call(
        _head_body,
        grid=(grid,),
        in_specs=[pl.BlockSpec((BM, 128), lambda i: (i, 0)),
                  pl.BlockSpec((128, 128), lambda i: (0, 0)),
                  pl.BlockSpec((128,), lambda i: (0,)),
                  pl.BlockSpec((128, 8), lambda i: (0, 0)),
                  pl.BlockSpec((8,), lambda i: (0,))],
        out_specs=pl.BlockSpec((BM, 8), lambda i: (i, 0)),
        out_shape=jax.ShapeDtypeStruct((n, 8), jnp.float32),
    )(x, wp, bp, wo, bo)


def _edge_sc(n, e, lo):
    C = 32
    NH = 5120
    nhv = min(NH, n - lo)
    ept = e // 16
    nchunk = ept // C
    tpr = NH // 16
    nfin = tpr // C
    na = NH + 16
    mesh = plsc.VectorSubcoreMesh(core_axis_name="c", subcore_axis_name="s")

    @functools.partial(
        pl.kernel, mesh=mesh,
        compiler_params=pltpu.CompilerParams(needs_layout_passes=False),
        out_type=[jax.ShapeDtypeStruct((2 * nhv, 128), jnp.float32),
                  jax.ShapeDtypeStruct((2 * nhv, 16), jnp.float32)],
        scratch_types=[
            pltpu.VMEM((C,), jnp.int32),
            pltpu.VMEM((C,), jnp.int32),
            pltpu.VMEM((C,), jnp.int32),
            pltpu.VMEM((C,), jnp.int32),
            pltpu.VMEM((C,), jnp.int32),
            pltpu.VMEM((C,), jnp.int32),
            pltpu.VMEM((C, 128), jnp.float32),
            pltpu.VMEM((C, 128), jnp.float32),
            pltpu.VMEM((C, 128), jnp.float32),
            pltpu.VMEM((C, 128), jnp.float32),
            pltpu.VMEM((C, 16), jnp.float32),
            pltpu.VMEM_SHARED((na, 128), jnp.float32),
            pltpu.VMEM_SHARED((na, 16), jnp.float32),
            pltpu.SemaphoreType.DMA,
            pltpu.SemaphoreType.DMA,
            pltpu.SemaphoreType.DMA,
        ],
    )
    def k(qt, kt, vt, dsth, srch, eth, aggn, aggd,
          dstb, srcb, etb, idxq, idxs, sidx, qb, kb, vb, numb, denb,
          num_sh, den_sh, s1, s2, s3):
        c = lax.axis_index("c")
        s = lax.axis_index("s")
        cn = c * n
        cexp = c * nhv
        zero16 = jnp.zeros((16,), jnp.float32)

        def zrow(r, carry):
            for j in range(8):
                numb[r, pl.ds(j * 16, 16)] = zero16
            denb[r, pl.ds(0, 16)] = zero16
            return carry
        lax.fori_loop(0, C, zrow, 0)
        r0 = s * tpr
        for j in range(nfin):
            pltpu.sync_copy(numb, num_sh.at[pl.ds(r0 + j * C, C)])
            pltpu.sync_copy(denb, den_sh.at[pl.ds(r0 + j * C, C)])
        plsc.subcore_barrier()

        e0 = s * ept

        def chunk(i, carry):
            base = e0 + i * C
            pltpu.sync_copy(dsth.at[pl.ds(base, C)], dstb)
            pltpu.sync_copy(srch.at[pl.ds(base, C)], srcb)
            pltpu.sync_copy(eth.at[pl.ds(base, C)], etb)
            for j in range(C // 16):
                sl = pl.ds(j * 16, 16)
                idxq[sl] = dstb[sl] + cn
                idxs[sl] = srcb[sl] + cn
                sidx[sl] = jnp.where(etb[sl] == c, dstb[sl], n)
            cp1 = pltpu.async_copy(qt.at[idxq], qb, s1)
            cp2 = pltpu.async_copy(kt.at[idxs], kb, s2)
            cp3 = pltpu.async_copy(vt.at[idxs], vb, s3)
            cp1.wait()
            cp2.wait()
            cp3.wait()

            def group(g, carry2):
                rows = lax.broadcasted_iota(jnp.int32, (16,), 0) + g * 16
                for h in range(HEADS):
                    sacc = jnp.zeros((16,), jnp.float32)
                    for d in range(DH):
                        cvec = jnp.full((16,), h * DH + d, jnp.int32)
                        qc = plsc.load_gather(qb, [rows, cvec])
                        kc = plsc.load_gather(kb, [rows, cvec])
                        sacc = sacc + qc * kc
                    ex = jnp.exp(sacc)
                    for d in range(DH):
                        cvec = jnp.full((16,), h * DH + d, jnp.int32)
                        vc = plsc.load_gather(vb, [rows, cvec])
                        plsc.store_scatter(numb, [rows, cvec], vc * ex)
                    plsc.store_scatter(
                        denb, [rows, jnp.full((16,), h * 4, jnp.int32)], ex)
                return carry2
            lax.fori_loop(0, C // 16, group, 0)
            pltpu.sync_copy(numb, num_sh.at[sidx], add=True)
            pltpu.sync_copy(denb, den_sh.at[sidx], add=True)
            return carry
        lax.fori_loop(0, nchunk, chunk, 0)
        plsc.subcore_barrier()

        nb = (nhv // C) * C
        npart = nhv % C

        def fin(j, carry):
            rf = s * tpr + j * C
            pltpu.sync_copy(num_sh.at[pl.ds(rf, C)], numb)
            pltpu.sync_copy(den_sh.at[pl.ds(rf, C)], denb)

            @pl.when(rf + C <= nhv)
            def _full():
                pltpu.sync_copy(numb, aggn.at[pl.ds(cexp + rf, C)])
                pltpu.sync_copy(denb, aggd.at[pl.ds(cexp + rf, C)])

            if npart:
                @pl.when(rf == nb)
                def _part():
                    pltpu.sync_copy(numb.at[pl.ds(0, npart)],
                                    aggn.at[pl.ds(cexp + rf, npart)])
                    pltpu.sync_copy(denb.at[pl.ds(0, npart)],
                                    aggd.at[pl.ds(cexp + rf, npart)])
            return carry
        lax.fori_loop(0, nfin, fin, 0)

    return k


def kernel(LM_embedding, x_numerical, x_categorical, edge_index, edge_type, params):
    p = params
    n = LM_embedding.shape[0]
    e = edge_index.shape[1]
    scale = 1.0 / jnp.sqrt(float(DH))

    x = _encoder(LM_embedding, x_numerical, x_categorical, p)

    e2 = ((e + 1023) // 1024) * 1024
    src = jnp.pad(edge_index[0], (0, e2 - e))
    dst = jnp.pad(edge_index[1], (0, e2 - e))
    etp = jnp.pad(edge_type, (0, e2 - e), constant_values=2)
    edge_lo = _edge_sc(n, e2, 0)
    edge_hi = _edge_sc(n, e2, 5120)
    nlo, nhi = 5120, n - 5120
    bmat = (jnp.arange(16)[:, None]
            == (jnp.arange(128)[None, :] // DH) * 4).astype(jnp.float32)

    for i in range(N_LAYERS):
        qt, kt, vt, skip = _projections(
            x,
            p["Wq"][i] * scale, p["bq"][i] * scale,
            p["Wk"][i], p["bk"][i],
            p["Wv"][i], p["bv"][i],
            p["Wskip"][i], p["bskip"][i])
        qtf = qt.reshape(2 * n, 128)
        ktf = kt.reshape(2 * n, 128)
        vtf = vt.reshape(2 * n, 128)
        an0, ad0 = edge_lo(qtf, ktf, vtf, dst, src, etp)
        an1, ad1 = edge_hi(qtf, ktf, vtf, dst, src, etp)
        an0 = an0.reshape(N_REL, nlo, 128)
        an1 = an1.reshape(N_REL, nhi, 128)
        ad0 = ad0.reshape(N_REL, nlo, 16)
        ad1 = ad1.reshape(N_REL, nhi, 16)
        agg_n = jnp.concatenate([an0, an1], axis=1)
        agg_d = jnp.concatenate([ad0, ad1], axis=1)
        z, ps = _combine(agg_n, agg_d, bmat, skip,
                         p["Wsem"][i], p["bsem"][i])
        x = _mix(z, ps, p["asem"][i])

    wo = jnp.pad(p["W_out"], ((0, 0), (0, 6)))
    bo = jnp.pad(p["b_out"], (0, 6))
    out = _head(x, p["W_pool"], p["b_pool"], wo, bo)
    return out[:, :2]


# speedup vs baseline: 4.2125x; 1.0859x over previous
import functools

import jax
import jax.numpy as jnp
from jax import lax
from jax.experimental import pallas as pl
from jax.experimental.pallas import tpu as pltpu
from jax.experimental.pallas import tpu_sc as plsc

N_LAYERS = 2
N_REL = 2
HEADS = 4
SEM_HEADS = 4
DH = 32
BM = 1000


def _enc_body(lm, xn, xc, wlm, blm, wnum, bnum, wcat, bcat, wmlp, bmlp, o):
    r = lambda t: jnp.maximum(t, 0.0)
    hl = r(jnp.dot(lm[...], wlm[...], preferred_element_type=jnp.float32) + blm[...])
    hn = r(jnp.dot(xn[...], wnum[...], preferred_element_type=jnp.float32) + bnum[...])
    hc = r(jnp.dot(xc[...], wcat[...], preferred_element_type=jnp.float32) + bcat[...])
    acc = jnp.dot(hl, wmlp[0:64, :], preferred_element_type=jnp.float32)
    acc += jnp.dot(hn, wmlp[64:96, :], preferred_element_type=jnp.float32)
    acc += jnp.dot(hc, wmlp[96:128, :], preferred_element_type=jnp.float32)
    o[...] = r(acc + bmlp[...])


def _encoder(LM, xn, xc, p):
    n = LM.shape[0]
    grid = n // BM
    return pl.pallas_call(
        _enc_body,
        grid=(grid,),
        in_specs=[
            pl.BlockSpec((BM, LM.shape[1]), lambda i: (i, 0)),
            pl.BlockSpec((BM, xn.shape[1]), lambda i: (i, 0)),
            pl.BlockSpec((BM, xc.shape[1]), lambda i: (i, 0)),
            pl.BlockSpec(p["W_lm"].shape, lambda i: (0, 0)),
            pl.BlockSpec(p["b_lm"].shape, lambda i: (0,)),
            pl.BlockSpec(p["W_num"].shape, lambda i: (0, 0)),
            pl.BlockSpec(p["b_num"].shape, lambda i: (0,)),
            pl.BlockSpec(p["W_cat"].shape, lambda i: (0, 0)),
            pl.BlockSpec(p["b_cat"].shape, lambda i: (0,)),
            pl.BlockSpec(p["W_mlp"].shape, lambda i: (0, 0)),
            pl.BlockSpec(p["b_mlp"].shape, lambda i: (0,)),
        ],
        out_specs=pl.BlockSpec((BM, 128), lambda i: (i, 0)),
        out_shape=jax.ShapeDtypeStruct((n, 128), jnp.float32),
    )(LM, xn, xc, p["W_lm"], p["b_lm"], p["W_num"], p["b_num"],
      p["W_cat"], p["b_cat"], p["W_mlp"], p["b_mlp"])


def _proj_body(x, wq, bq, wk, bk, wv, bv, ws, bs, qo, ko, vo, so):
    xv = x[...]
    for r in range(N_REL):
        qo[r] = jnp.dot(xv, wq[r], preferred_element_type=jnp.float32) + bq[r]
        ko[r] = jnp.dot(xv, wk[r], preferred_element_type=jnp.float32) + bk[r]
        vo[r] = jnp.dot(xv, wv[r], preferred_element_type=jnp.float32) + bv[r]
        so[r] = jnp.dot(xv, ws[r], preferred_element_type=jnp.float32) + bs[r]


def _projections(x, wq, bq, wk, bk, wv, bv, ws, bs):
    n = x.shape[0]
    grid = n // BM
    wspec = pl.BlockSpec((N_REL, 128, 128), lambda i: (0, 0, 0))
    bspec = pl.BlockSpec((N_REL, 128), lambda i: (0, 0))
    ospec = pl.BlockSpec((N_REL, BM, 128), lambda i: (0, i, 0))
    oshape = jax.ShapeDtypeStruct((N_REL, n, 128), jnp.float32)
    return pl.pallas_call(
        _proj_body,
        grid=(grid,),
        in_specs=[pl.BlockSpec((BM, 128), lambda i: (i, 0)),
                  wspec, bspec, wspec, bspec, wspec, bspec, wspec, bspec],
        out_specs=[ospec, ospec, ospec, ospec],
        out_shape=[oshape, oshape, oshape, oshape],
    )(x, wq, bq, wk, bk, wv, bv, ws, bs)


def _combine_body(num, den, bmat, skip, wsem, bsem, z, ps):
    i = pl.program_id(0)

    @pl.when(i == 0)
    def _init():
        ps[...] = jnp.zeros_like(ps)

    sums = []
    for r in range(N_REL):
        db = jnp.dot(den[r], bmat[...], preferred_element_type=jnp.float32)
        zr = num[r] / (db + 1e-16) + skip[r]
        z[r] = zr
        t = jnp.tanh(jnp.dot(zr, wsem[...], preferred_element_type=jnp.float32)
                     + bsem[...])
        sums.append(jnp.sum(t, axis=0))
    ps[...] += jnp.stack(sums, axis=0)


def _combine(num, den, bmat, skip, wsem, bsem):
    n = num.shape[1]
    grid = n // BM
    return pl.pallas_call(
        _combine_body,
        grid=(grid,),
        in_specs=[pl.BlockSpec((N_REL, BM, 128), lambda i: (0, i, 0)),
                  pl.BlockSpec((N_REL, BM, 16), lambda i: (0, i, 0)),
                  pl.BlockSpec((16, 128), lambda i: (0, 0)),
                  pl.BlockSpec((N_REL, BM, 128), lambda i: (0, i, 0)),
                  pl.BlockSpec((128, 128), lambda i: (0, 0)),
                  pl.BlockSpec((128,), lambda i: (0,))],
        out_specs=[pl.BlockSpec((N_REL, BM, 128), lambda i: (0, i, 0)),
                   pl.BlockSpec((N_REL, 128), lambda i: (0, 0))],
        out_shape=[jax.ShapeDtypeStruct((N_REL, n, 128), jnp.float32),
                   jax.ShapeDtypeStruct((N_REL, 128), jnp.float32)],
    )(num, den, bmat, skip, wsem, bsem)


def _mix_body(z, ps, asem, o):
    n_total = o.shape[0] * pl.num_programs(0)
    score = jnp.dot(ps[...], asem[...],
                    preferred_element_type=jnp.float32) / float(n_total)
    m = jnp.max(score, axis=0, keepdims=True)
    e = jnp.exp(score - m)
    beta = e / jnp.sum(e, axis=0, keepdims=True)
    c0 = jnp.sum(beta[0:1, :]) / float(SEM_HEADS)
    c1 = jnp.sum(beta[1:2, :]) / float(SEM_HEADS)
    o[...] = jnp.maximum(z[0] * c0 + z[1] * c1, 0.0)


def _mix(z, ps, asem):
    n = z.shape[1]
    grid = n // BM
    return pl.pallas_call(
        _mix_body,
        grid=(grid,),
        in_specs=[pl.BlockSpec((N_REL, BM, 128), lambda i: (0, i, 0)),
                  pl.BlockSpec((N_REL, 128), lambda i: (0, 0)),
                  pl.BlockSpec((128, SEM_HEADS), lambda i: (0, 0))],
        out_specs=pl.BlockSpec((BM, 128), lambda i: (i, 0)),
        out_shape=jax.ShapeDtypeStruct((n, 128), jnp.float32),
    )(z, ps, asem)


def _head_body(x, wp, bp, wo, bo, o):
    h = jnp.maximum(jnp.dot(x[...], wp[...], preferred_element_type=jnp.float32)
                    + bp[...], 0.0)
    o[...] = jnp.dot(h, wo[...], preferred_element_type=jnp.float32) + bo[...]


def _head(x, wp, bp, wo, bo):
    n = x.shape[0]
    grid = n // BM
    return pl.pallas_call(
        _head_body,
        grid=(grid,),
        in_specs=[pl.BlockSpec((BM, 128), lambda i: (i, 0)),
                  pl.BlockSpec((128, 128), lambda i: (0, 0)),
                  pl.BlockSpec((128,), lambda i: (0,)),
                  pl.BlockSpec((128, 8), lambda i: (0, 0)),
                  pl.BlockSpec((8,), lambda i: (0,))],
        out_specs=pl.BlockSpec((BM, 8), lambda i: (i, 0)),
        out_shape=jax.ShapeDtypeStruct((n, 8), jnp.float32),
    )(x, wp, bp, wo, bo)


def _edge_sc(n, e, lo):
    C = 64
    NH = 5120
    nhv = min(NH, n - lo)
    ept = e // 16
    nchunk = ept // C
    tpr = NH // 16
    nfin = tpr // C
    na = NH + 16
    mesh = plsc.VectorSubcoreMesh(core_axis_name="c", subcore_axis_name="s")

    @functools.partial(
        pl.kernel, mesh=mesh,
        compiler_params=pltpu.CompilerParams(needs_layout_passes=False),
        out_type=[jax.ShapeDtypeStruct((2 * nhv, 128), jnp.float32),
                  jax.ShapeDtypeStruct((2 * nhv, 16), jnp.float32)],
        scratch_types=[
            pltpu.VMEM((C,), jnp.int32),
            pltpu.VMEM((C,), jnp.int32),
            pltpu.VMEM((C,), jnp.int32),
            pltpu.VMEM((C,), jnp.int32),
            pltpu.VMEM((C,), jnp.int32),
            pltpu.VMEM((C,), jnp.int32),
            pltpu.VMEM((C, 128), jnp.float32),
            pltpu.VMEM((C, 128), jnp.float32),
            pltpu.VMEM((C, 128), jnp.float32),
            pltpu.VMEM((C, 128), jnp.float32),
            pltpu.VMEM((C, 16), jnp.float32),
            pltpu.VMEM_SHARED((na, 128), jnp.float32),
            pltpu.VMEM_SHARED((na, 16), jnp.float32),
            pltpu.SemaphoreType.DMA,
            pltpu.SemaphoreType.DMA,
            pltpu.SemaphoreType.DMA,
        ],
    )
    def k(qt, kt, vt, dsth, srch, eth, aggn, aggd,
          dstb, srcb, etb, idxq, idxs, sidx, qb, kb, vb, numb, denb,
          num_sh, den_sh, s1, s2, s3):
        c = lax.axis_index("c")
        s = lax.axis_index("s")
        cn = c * n
        cexp = c * nhv
        zero16 = jnp.zeros((16,), jnp.float32)

        def zrow(r, carry):
            for j in range(8):
                numb[r, pl.ds(j * 16, 16)] = zero16
            denb[r, pl.ds(0, 16)] = zero16
            return carry
        lax.fori_loop(0, C, zrow, 0)
        r0 = s * tpr
        for j in range(nfin):
            pltpu.sync_copy(numb, num_sh.at[pl.ds(r0 + j * C, C)])
            pltpu.sync_copy(denb, den_sh.at[pl.ds(r0 + j * C, C)])
        plsc.subcore_barrier()

        e0 = s * ept

        def chunk(i, carry):
            base = e0 + i * C
            pltpu.sync_copy(dsth.at[pl.ds(base, C)], dstb)
            pltpu.sync_copy(srch.at[pl.ds(base, C)], srcb)
            pltpu.sync_copy(eth.at[pl.ds(base, C)], etb)
            for j in range(C // 16):
                sl = pl.ds(j * 16, 16)
                idxq[sl] = dstb[sl] + cn
                idxs[sl] = srcb[sl] + cn
                sidx[sl] = jnp.where(etb[sl] == c, dstb[sl], n)
            cp1 = pltpu.async_copy(qt.at[idxq], qb, s1)
            cp2 = pltpu.async_copy(kt.at[idxs], kb, s2)
            cp3 = pltpu.async_copy(vt.at[idxs], vb, s3)
            cp1.wait()
            cp2.wait()
            cp3.wait()

            def group(g, carry2):
                rows = lax.broadcasted_iota(jnp.int32, (16,), 0) + g * 16
                for h in range(HEADS):
                    sacc = jnp.zeros((16,), jnp.float32)
                    for d in range(DH):
                        cvec = jnp.full((16,), h * DH + d, jnp.int32)
                        qc = plsc.load_gather(qb, [rows, cvec])
                        kc = plsc.load_gather(kb, [rows, cvec])
                        sacc = sacc + qc * kc
                    ex = jnp.exp(sacc)
                    for d in range(DH):
                        cvec = jnp.full((16,), h * DH + d, jnp.int32)
                        vc = plsc.load_gather(vb, [rows, cvec])
                        plsc.store_scatter(numb, [rows, cvec], vc * ex)
                    plsc.store_scatter(
                        denb, [rows, jnp.full((16,), h * 4, jnp.int32)], ex)
                return carry2
            lax.fori_loop(0, C // 16, group, 0)
            pltpu.sync_copy(numb, num_sh.at[sidx], add=True)
            pltpu.sync_copy(denb, den_sh.at[sidx], add=True)
            return carry
        lax.fori_loop(0, nchunk, chunk, 0)
        plsc.subcore_barrier()

        nb = (nhv // C) * C
        npart = nhv % C

        def fin(j, carry):
            rf = s * tpr + j * C
            pltpu.sync_copy(num_sh.at[pl.ds(rf, C)], numb)
            pltpu.sync_copy(den_sh.at[pl.ds(rf, C)], denb)

            @pl.when(rf + C <= nhv)
            def _full():
                pltpu.sync_copy(numb, aggn.at[pl.ds(cexp + rf, C)])
                pltpu.sync_copy(denb, aggd.at[pl.ds(cexp + rf, C)])

            if npart:
                @pl.when(rf == nb)
                def _part():
                    pltpu.sync_copy(numb.at[pl.ds(0, npart)],
                                    aggn.at[pl.ds(cexp + rf, npart)])
                    pltpu.sync_copy(denb.at[pl.ds(0, npart)],
                                    aggd.at[pl.ds(cexp + rf, npart)])
            return carry
        lax.fori_loop(0, nfin, fin, 0)

    return k


def kernel(LM_embedding, x_numerical, x_categorical, edge_index, edge_type, params):
    p = params
    n = LM_embedding.shape[0]
    e = edge_index.shape[1]
    scale = 1.0 / jnp.sqrt(float(DH))

    x = _encoder(LM_embedding, x_numerical, x_categorical, p)

    e2 = ((e + 1023) // 1024) * 1024
    src = jnp.pad(edge_index[0], (0, e2 - e))
    dst = jnp.pad(edge_index[1], (0, e2 - e))
    etp = jnp.pad(edge_type, (0, e2 - e), constant_values=2)
    edge_lo = _edge_sc(n, e2, 0)
    edge_hi = _edge_sc(n, e2, 5120)
    nlo, nhi = 5120, n - 5120
    bmat = (jnp.arange(16)[:, None]
            == (jnp.arange(128)[None, :] // DH) * 4).astype(jnp.float32)

    for i in range(N_LAYERS):
        qt, kt, vt, skip = _projections(
            x,
            p["Wq"][i] * scale, p["bq"][i] * scale,
            p["Wk"][i], p["bk"][i],
            p["Wv"][i], p["bv"][i],
            p["Wskip"][i], p["bskip"][i])
        qtf = qt.reshape(2 * n, 128)
        ktf = kt.reshape(2 * n, 128)
        vtf = vt.reshape(2 * n, 128)
        an0, ad0 = edge_lo(qtf, ktf, vtf, dst, src, etp)
        an1, ad1 = edge_hi(qtf, ktf, vtf, dst, src, etp)
        an0 = an0.reshape(N_REL, nlo, 128)
        an1 = an1.reshape(N_REL, nhi, 128)
        ad0 = ad0.reshape(N_REL, nlo, 16)
        ad1 = ad1.reshape(N_REL, nhi, 16)
        agg_n = jnp.concatenate([an0, an1], axis=1)
        agg_d = jnp.concatenate([ad0, ad1], axis=1)
        z, ps = _combine(agg_n, agg_d, bmat, skip,
                         p["Wsem"][i], p["bsem"][i])
        x = _mix(z, ps, p["asem"][i])

    wo = jnp.pad(p["W_out"], ((0, 0), (0, 6)))
    bo = jnp.pad(p["b_out"], (0, 6))
    out = _head(x, p["W_pool"], p["b_pool"], wo, bo)
    return out[:, :2]
